# baseline (speedup 1.0000x reference)
import functools

import jax
import jax.numpy as jnp
from jax import lax
from jax.experimental import pallas as pl
from jax.experimental.pallas import tpu as pltpu
from jax.experimental.pallas import tpu_sc as plsc

DIM = 64
NC = 2
NS = 16
NW = NC * NS
CHUNK = 128
LANES = 16
SCALE = 8.0


def _emb_body(idx_hbm, table_hbm, out_hbm, idx_v, rows_v, gsem):
    nchunk = idx_v.shape[0]
    wid = lax.axis_index("s") * NC + lax.axis_index("c")
    base = wid * (nchunk * CHUNK)

    pltpu.sync_copy(idx_hbm.at[wid], idx_v)

    def chunk_body(j, carry):
        pltpu.async_copy(table_hbm.at[idx_v.at[j]], rows_v, gsem).wait()

        def row_body(r, c):
            for d in range(DIM // LANES):
                sl = pl.ds(d * LANES, LANES)
                rows_v[r, sl] = rows_v[r, sl] * SCALE
            return c

        lax.fori_loop(0, CHUNK, row_body, 0, unroll=2)

        pltpu.sync_copy(rows_v, out_hbm.at[pl.ds(base + j * CHUNK, CHUNK)])
        return carry

    lax.fori_loop(0, nchunk, chunk_body, 0)


def kernel(input_vec, table):
    b0, b1 = input_vec.shape
    total = b0 * b1
    per_w = total // NW
    nchunk = per_w // CHUNK
    idx = input_vec.astype(jnp.int32).reshape(NW, nchunk, CHUNK)

    run = functools.partial(
        pl.kernel,
        mesh=plsc.VectorSubcoreMesh(core_axis_name="c", subcore_axis_name="s"),
        out_type=jax.ShapeDtypeStruct((total, DIM), jnp.float32),
        scratch_types=[
            pltpu.VMEM((nchunk, CHUNK), jnp.int32),
            pltpu.VMEM((CHUNK, DIM), jnp.float32),
            pltpu.SemaphoreType.DMA,
        ],
        compiler_params=pltpu.CompilerParams(use_tc_tiling_on_sc=False),
    )(_emb_body)
    out = run(idx, table)
    return out.reshape(b0, b1, DIM)


# speedup vs baseline: 1.0342x; 1.0342x over previous
import functools

import jax
import jax.numpy as jnp
from jax import lax
from jax.experimental import pallas as pl
from jax.experimental.pallas import tpu as pltpu
from jax.experimental.pallas import tpu_sc as plsc

DIM = 64
NC = 2
NS = 16
NW = NC * NS
CHUNK = 128
LANES = 16
NBUF = 5
SCALE = 8.0


def _emb_body(idx_hbm, table_hbm, out_hbm, idx_v, rows_v, obuf_v, gsem, osem):
    nchunk = idx_v.shape[0]
    wid = lax.axis_index("s") * NC + lax.axis_index("c")
    base = wid * (nchunk * CHUNK)

    pltpu.sync_copy(idx_hbm.at[wid], idx_v)

    def fire_gather(j, b):
        pltpu.async_copy(table_hbm.at[idx_v.at[j]], rows_v.at[b], gsem)

    def wait_gather(j, b):
        pltpu.make_async_copy(table_hbm.at[idx_v.at[j]], rows_v.at[b], gsem).wait()

    def fire_out(j, b):
        pltpu.async_copy(obuf_v.at[b], out_hbm.at[pl.ds(base + j * CHUNK, CHUNK)], osem)

    def wait_out(b):
        pltpu.make_async_copy(obuf_v.at[b], out_hbm.at[pl.ds(base, CHUNK)], osem).wait()

    def scale_chunk(b):
        def row_body(r, c):
            for d in range(DIM // LANES):
                sl = pl.ds(d * LANES, LANES)
                obuf_v[b, r, sl] = rows_v[b, r, sl] * SCALE
            return c

        lax.fori_loop(0, CHUNK, row_body, 0, unroll=4)

    for b in range(NBUF):
        fire_gather(b, b)

    def outer(g, carry):
        for b in range(NBUF):
            j = g + b
            wait_gather(j, b)

            @pl.when(g > 0)
            def _():
                wait_out(b)

            scale_chunk(b)
            fire_out(j, b)

            @pl.when(j + NBUF < nchunk)
            def _():
                fire_gather(j + NBUF, b)

        return carry

    lax.fori_loop(0, nchunk // NBUF, lambda t, c: outer(t * NBUF, c), 0)

    for b in range(NBUF):
        wait_out(b)


def kernel(input_vec, table):
    b0, b1 = input_vec.shape
    total = b0 * b1
    per_w = total // NW
    nchunk = per_w // CHUNK
    idx = input_vec.astype(jnp.int32).reshape(NW, nchunk, CHUNK)

    run = functools.partial(
        pl.kernel,
        mesh=plsc.VectorSubcoreMesh(core_axis_name="c", subcore_axis_name="s"),
        out_type=jax.ShapeDtypeStruct((total, DIM), jnp.float32),
        scratch_types=[
            pltpu.VMEM((nchunk, CHUNK), jnp.int32),
            pltpu.VMEM((NBUF, CHUNK, DIM), jnp.float32),
            pltpu.VMEM((NBUF, CHUNK, DIM), jnp.float32),
            pltpu.SemaphoreType.DMA,
            pltpu.SemaphoreType.DMA,
        ],
        compiler_params=pltpu.CompilerParams(use_tc_tiling_on_sc=False),
    )(_emb_body)
    out = run(idx, table)
    return out.reshape(b0, b1, DIM)
